# Initial kernel scaffold; baseline (speedup 1.0000x reference)
import functools

import jax
import jax.numpy as jnp
from jax import lax
from jax.experimental import pallas as pl
from jax.experimental.pallas import tpu as pltpu
from jax.experimental.pallas import tpu_sc as plsc

NUM_SC = 2
NUM_TILES = 16
NUM_W = NUM_SC * NUM_TILES
CHUNK = 128
ZROWS = 64


def _sc_aggregate(x, src, dst, n_pad):
    n, d = x.shape
    e_pad = src.shape[0]
    ept = e_pad // NUM_W
    nchunks = ept // CHUNK
    rpt = n_pad // NUM_TILES

    mesh = plsc.VectorSubcoreMesh(core_axis_name="c", subcore_axis_name="s")

    @functools.partial(
        pl.kernel,
        out_type=(
            jax.ShapeDtypeStruct((NUM_SC, n_pad, d), jnp.float32),
            jax.ShapeDtypeStruct((NUM_SC, n_pad), jnp.float32),
        ),
        mesh=mesh,
        scratch_types=[
            pltpu.VMEM((CHUNK,), jnp.int32),
            pltpu.VMEM((CHUNK,), jnp.int32),
            pltpu.VMEM((CHUNK, d), jnp.float32),
            pltpu.VMEM((CHUNK,), jnp.float32),
            pltpu.VMEM((ZROWS, d), jnp.float32),
            pltpu.VMEM((n_pad // NUM_TILES,), jnp.float32),
            pltpu.VMEM_SHARED((n_pad, d), jnp.float32),
            pltpu.VMEM_SHARED((n_pad,), jnp.float32),
            pltpu.SemaphoreType.DMA,
        ],
    )
    def agg(x_hbm, src_hbm, dst_hbm, acc_out, cnt_out,
            src_v, dst_v, rows_v, ones_v, zrow_v, zcnt_v, acc_sh, cnt_sh,
            sem):
        c = lax.axis_index("c")
        s = lax.axis_index("s")
        wid = s * NUM_SC + c
        row0 = s * rpt

        zeros16 = jnp.zeros((16,), jnp.float32)
        ones16 = jnp.ones((16,), jnp.float32)

        def fill_zrow(r, carry):
            for j in range(d // 16):
                zrow_v[r, pl.ds(j * 16, 16)] = zeros16
            return carry
        lax.fori_loop(0, ZROWS, fill_zrow, 0)

        def fill_zcnt(i, carry):
            zcnt_v[pl.ds(i * 16, 16)] = zeros16
            return carry
        lax.fori_loop(0, rpt // 16, fill_zcnt, 0)

        for j in range(CHUNK // 16):
            ones_v[pl.ds(j * 16, 16)] = ones16

        def zero_acc(k, carry):
            pltpu.sync_copy(zrow_v, acc_sh.at[pl.ds(row0 + k * ZROWS, ZROWS)])
            return carry
        lax.fori_loop(0, rpt // ZROWS, zero_acc, 0)
        pltpu.sync_copy(zcnt_v, cnt_sh.at[pl.ds(row0, rpt)])

        plsc.subcore_barrier()

        def chunk_body(i, carry):
            base = wid * ept + i * CHUNK
            pltpu.sync_copy(src_hbm.at[pl.ds(base, CHUNK)], src_v)
            pltpu.sync_copy(dst_hbm.at[pl.ds(base, CHUNK)], dst_v)
            pltpu.async_copy(x_hbm.at[src_v], rows_v, sem).wait()
            pltpu.sync_copy(rows_v, acc_sh.at[dst_v], add=True)
            pltpu.sync_copy(ones_v, cnt_sh.at[dst_v], add=True)
            return carry
        lax.fori_loop(0, nchunks, chunk_body, 0)

        plsc.subcore_barrier()

        pltpu.sync_copy(acc_sh.at[pl.ds(row0, rpt)],
                        acc_out.at[c, pl.ds(row0, rpt)])
        pltpu.sync_copy(cnt_sh.at[pl.ds(row0, rpt)],
                        cnt_out.at[c, pl.ds(row0, rpt)])

    return agg(x, src, dst)


def _tc_finish(acc, cnt, x, w_l, w_r, blk):
    n, d = x.shape
    n_pad = acc.shape[1]

    def body(acc_ref, cnt_ref, x_ref, wl_ref, wr_ref, out_ref):
        i = pl.program_id(0)
        a = acc_ref[0] + acc_ref[1]
        ct = (cnt_ref[0, pl.ds(i * blk, blk)]
              + cnt_ref[1, pl.ds(i * blk, blk)])
        ct = jnp.maximum(ct, 1.0)
        mean = a / ct[:, None]
        dn = (((1,), (1,)), ((), ()))
        out_ref[...] = (
            lax.dot_general(mean, wl_ref[...], dn,
                            preferred_element_type=jnp.float32)
            + lax.dot_general(x_ref[...], wr_ref[...], dn,
                              preferred_element_type=jnp.float32))

    return pl.pallas_call(
        body,
        out_shape=jax.ShapeDtypeStruct((n, d), jnp.float32),
        grid=(n // blk,),
        in_specs=[
            pl.BlockSpec((NUM_SC, blk, d), lambda i: (0, i, 0)),
            pl.BlockSpec((NUM_SC, n_pad), lambda i: (0, 0)),
            pl.BlockSpec((blk, d), lambda i: (i, 0)),
            pl.BlockSpec((d, d), lambda i: (0, 0)),
            pl.BlockSpec((d, d), lambda i: (0, 0)),
        ],
        out_specs=pl.BlockSpec((blk, d), lambda i: (i, 0)),
    )(acc, cnt, x, w_l, w_r)


def kernel(x, edge_index, W_l, W_r):
    n, d = x.shape
    e = edge_index.shape[1]

    n_pad = -(-n // (NUM_TILES * ZROWS)) * (NUM_TILES * ZROWS)
    e_pad = -(-e // (NUM_W * CHUNK)) * (NUM_W * CHUNK)

    src = edge_index[0]
    dst = edge_index[1]
    if e_pad != e:
        pad = e_pad - e
        src = jnp.concatenate([src, jnp.zeros((pad,), jnp.int32)])
        dst = jnp.concatenate([dst, jnp.full((pad,), n_pad - 1, jnp.int32)])

    acc, cnt = _sc_aggregate(x, src, dst, n_pad)
    return _tc_finish(acc, cnt, x, W_l, W_r, blk=400)


# speedup vs baseline: 4.7226x; 4.7226x over previous
import functools

import jax
import jax.numpy as jnp
from jax import lax
from jax.experimental import pallas as pl
from jax.experimental.pallas import tpu as pltpu
from jax.experimental.pallas import tpu_sc as plsc

NUM_SC = 2
NUM_TILES = 16
NUM_W = NUM_SC * NUM_TILES
CHUNK = 128
ZROWS = 64


def _sc_aggregate(x, src, dst, n_pad):
    n, d = x.shape
    e_pad = src.shape[0]
    ept = e_pad // NUM_W
    nchunks = ept // CHUNK
    rpt = n_pad // NUM_TILES

    mesh = plsc.VectorSubcoreMesh(core_axis_name="c", subcore_axis_name="s")

    @functools.partial(
        pl.kernel,
        out_type=(
            jax.ShapeDtypeStruct((NUM_SC, n_pad, d), jnp.float32),
            jax.ShapeDtypeStruct((NUM_SC, n_pad), jnp.float32),
        ),
        mesh=mesh,
        scratch_types=[
            pltpu.VMEM((CHUNK,), jnp.int32),
            pltpu.VMEM((CHUNK,), jnp.int32),
            pltpu.VMEM((CHUNK, d), jnp.float32),
            pltpu.VMEM((CHUNK,), jnp.float32),
            pltpu.VMEM((ZROWS, d), jnp.float32),
            pltpu.VMEM((n_pad // NUM_TILES,), jnp.float32),
            pltpu.VMEM_SHARED((n_pad, d), jnp.float32),
            pltpu.VMEM_SHARED((n_pad,), jnp.float32),
            pltpu.SemaphoreType.DMA,
        ],
    )
    def agg(x_hbm, src_hbm, dst_hbm, acc_out, cnt_out,
            src_v, dst_v, rows_v, ones_v, zrow_v, zcnt_v, acc_sh, cnt_sh,
            sem):
        c = lax.axis_index("c")
        s = lax.axis_index("s")
        wid = s * NUM_SC + c
        row0 = s * rpt

        zeros16 = jnp.zeros((16,), jnp.float32)
        ones16 = jnp.ones((16,), jnp.float32)

        def fill_zrow(r, carry):
            for j in range(d // 16):
                zrow_v[r, pl.ds(j * 16, 16)] = zeros16
            return carry
        lax.fori_loop(0, ZROWS, fill_zrow, 0)

        def fill_zcnt(i, carry):
            zcnt_v[pl.ds(i * 16, 16)] = zeros16
            return carry
        lax.fori_loop(0, rpt // 16, fill_zcnt, 0)

        for j in range(CHUNK // 16):
            ones_v[pl.ds(j * 16, 16)] = ones16

        def zero_acc(k, carry):
            pltpu.sync_copy(zrow_v, acc_sh.at[pl.ds(row0 + k * ZROWS, ZROWS)])
            return carry
        lax.fori_loop(0, rpt // ZROWS, zero_acc, 0)
        pltpu.sync_copy(zcnt_v, cnt_sh.at[pl.ds(row0, rpt)])

        plsc.subcore_barrier()

        def chunk_body(i, carry):
            base = wid * ept + i * CHUNK
            pltpu.sync_copy(src_hbm.at[pl.ds(base, CHUNK)], src_v)
            pltpu.sync_copy(dst_hbm.at[pl.ds(base, CHUNK)], dst_v)
            pltpu.async_copy(x_hbm.at[src_v], rows_v, sem).wait()
            pltpu.sync_copy(rows_v, acc_sh.at[dst_v], add=True)
            pltpu.sync_copy(ones_v, cnt_sh.at[dst_v], add=True)
            return carry
        lax.fori_loop(0, nchunks, chunk_body, 0)

        plsc.subcore_barrier()

        pltpu.sync_copy(acc_sh.at[pl.ds(row0, rpt)],
                        acc_out.at[c, pl.ds(row0, rpt)])
        pltpu.sync_copy(cnt_sh.at[pl.ds(row0, rpt)],
                        cnt_out.at[c, pl.ds(row0, rpt)])

    return agg(x, src, dst)


def _tc_finish(acc, cnt, x, w_l, w_r, blk):
    n, d = x.shape
    n_pad = acc.shape[1]

    def body(acc_ref, cnt_ref, x_ref, wl_ref, wr_ref, out_ref):
        a = acc_ref[0] + acc_ref[1]
        ct = cnt_ref[0] + cnt_ref[1]
        ct = jnp.maximum(ct, 1.0)
        mean = a / ct[:, None]
        dn = (((1,), (1,)), ((), ()))
        out_ref[...] = (
            lax.dot_general(mean, wl_ref[...], dn,
                            preferred_element_type=jnp.float32)
            + lax.dot_general(x_ref[...], wr_ref[...], dn,
                              preferred_element_type=jnp.float32))

    return pl.pallas_call(
        body,
        out_shape=jax.ShapeDtypeStruct((n_pad, d), jnp.float32),
        grid=(n_pad // blk,),
        in_specs=[
            pl.BlockSpec((NUM_SC, blk, d), lambda i: (0, i, 0)),
            pl.BlockSpec((NUM_SC, blk), lambda i: (0, i)),
            pl.BlockSpec((blk, d), lambda i: (i, 0)),
            pl.BlockSpec((d, d), lambda i: (0, 0)),
            pl.BlockSpec((d, d), lambda i: (0, 0)),
        ],
        out_specs=pl.BlockSpec((blk, d), lambda i: (i, 0)),
    )(acc, cnt, x, w_l, w_r)


def kernel(x, edge_index, W_l, W_r):
    n, d = x.shape
    e = edge_index.shape[1]

    n_pad = -(-n // (NUM_TILES * ZROWS)) * (NUM_TILES * ZROWS)
    e_pad = -(-e // (NUM_W * CHUNK)) * (NUM_W * CHUNK)

    src = edge_index[0]
    dst = edge_index[1]
    if e_pad != e:
        pad = e_pad - e
        src = jnp.concatenate([src, jnp.zeros((pad,), jnp.int32)])
        dst = jnp.concatenate([dst, jnp.full((pad,), n_pad - 1, jnp.int32)])

    acc, cnt = _sc_aggregate(x, src, dst, n_pad)
    x_pad = jnp.concatenate(
        [x, jnp.zeros((n_pad - n, d), jnp.float32)]) if n_pad != n else x
    out = _tc_finish(acc, cnt, x_pad, W_l, W_r, blk=1024)
    return out[:n]
